# baseline (speedup 1.0000x reference)
import functools
import math

import jax
import jax.numpy as jnp
from jax.experimental import pallas as pl
from jax.experimental.pallas import tpu as pltpu


def _denoiser_body(
    t_ref, traj_hbm, act_hbm,
    fw_ref, wt1_ref, bt1_ref, wt2_hbm, bt2_ref,
    bp_ref, wp_hbm, b0_ref, w0_hbm,
    lng_ref, lnb_ref, wr_hbm, br_ref,
    lnfg_ref, lnfb_ref, wf_hbm, bfin_ref,
    out_ref,
    traj_scr, act_scr, wt2_scr, wpt_scr, wpa_scr, w0_scr,
    wr_scr, wf_scr, sems,
    *, num_layers: int, eps: float, tb: int, d: int,
):
    f32 = jnp.float32
    bf16 = jnp.bfloat16
    step = pl.program_id(0)
    rows = pl.ds(step * tb, tb)

    streams = [
        (wt2_hbm, wt2_scr),
        (traj_hbm.at[rows, :], traj_scr),
        (act_hbm.at[rows, :], act_scr),
        (wp_hbm.at[:, 0:d, :], wpt_scr),
        (wp_hbm.at[:, d:, :], wpa_scr),
        (w0_hbm, w0_scr),
    ] + [(wr_hbm.at[i], wr_scr.at[i]) for i in range(num_layers)] \
        + [(wf_hbm, wf_scr)]
    for k, (src, dst) in enumerate(streams):
        pltpu.make_async_copy(src, dst, sems.at[k]).start()

    def wait(k):
        dst = streams[k][1]
        pltpu.make_async_copy(dst, dst, sems.at[k]).wait()

    def mm(a, w_ref):
        w = w_ref[...]
        w2 = w.reshape(-1, w.shape[-1]) if w.ndim == 3 else w
        return jnp.dot(a.astype(bf16), w2.astype(bf16),
                       preferred_element_type=f32)

    def ln_relu(v, g, b):
        mu = jnp.mean(v, axis=-1, keepdims=True)
        m2 = jnp.mean(v * v, axis=-1, keepdims=True)
        var = jnp.maximum(m2 - mu * mu, 0.0)
        return jnp.maximum((v - mu) * jax.lax.rsqrt(var + eps) * g + b, 0.0)

    t = t_ref[...]
    phase = t * (fw_ref[0] * (2.0 * math.pi))
    sc = jnp.concatenate(
        [jnp.sin(phase), jnp.sin(phase + 0.5 * math.pi)], axis=1)
    h1 = (t * wt1_ref[0]
          + jnp.dot(sc, wt1_ref[1:, :], preferred_element_type=f32)
          + bt1_ref[...])
    h1 = h1 * (1.0 / (1.0 + jnp.exp(-h1)))
    wait(0)
    te = mm(h1, wt2_scr)

    wait(1), wait(2), wait(3), wait(4)
    zx = mm(traj_scr[...], wpt_scr) + mm(act_scr[...], wpa_scr)
    bpt = bp_ref[...] + bt2_ref[...]

    ns = 8 if tb % 8 == 0 else 1
    sb = tb // ns
    sl = [slice(s * sb, (s + 1) * sb) for s in range(ns)]
    wait(5)
    h = [None] * ns
    for s in range(ns):
        zs = zx[sl[s]] + bpt + te[sl[s]]
        h[s] = mm(zs, w0_scr) + b0_ref[...]
    a = [None] * ns
    for i in range(num_layers):
        wait(6 + i)
        a[0] = ln_relu(h[0], lng_ref[i], lnb_ref[i])
        for s in range(1, ns):
            a[s] = ln_relu(h[s], lng_ref[i], lnb_ref[i])
            h[s - 1] = h[s - 1] + mm(a[s - 1], wr_scr.at[i]) + br_ref[i]
        h[ns - 1] = h[ns - 1] + mm(a[ns - 1], wr_scr.at[i]) + br_ref[i]
    wait(6 + num_layers)
    a[0] = ln_relu(h[0], lnfg_ref[...], lnfb_ref[...])
    for s in range(1, ns):
        a[s] = ln_relu(h[s], lnfg_ref[...], lnfb_ref[...])
        out_ref[sl[s - 1], :] = mm(a[s - 1], wf_scr) + bfin_ref[...]
    out_ref[sl[ns - 1], :] = mm(a[ns - 1], wf_scr) + bfin_ref[...]


def kernel(traj, act, timesteps, fourier_w, wt1, bt1, wt2, bt2, wp, bp,
           w0, b0, ln_g, ln_b, wr, br, lnf_g, lnf_b, wf, bf):
    f32 = jnp.float32
    b, hor, d = traj.shape
    dc = act.shape[-1]
    trajf = traj.reshape(b, hor * d)
    actf = act.reshape(b, hor * dc)
    t = timesteps.reshape(b, 1)

    E = wt2.shape[0]
    H = w0.shape[1]
    L = wr.shape[0]
    dout = wf.shape[1]
    half = fourier_w.shape[0]

    tb = 1024 if b >= 1024 else max(8, ((b + 7) // 8) * 8)
    b_pad = ((b + tb - 1) // tb) * tb
    if b_pad != b:
        trajf = jnp.pad(trajf, ((0, b_pad - b), (0, 0)))
        actf = jnp.pad(actf, ((0, b_pad - b), (0, 0)))
        t = jnp.pad(t, ((0, b_pad - b), (0, 0)))

    def row(v):
        return v.reshape(1, -1)

    weight_inputs = [
        fourier_w.reshape(1, half),
        wt1, row(bt1),
        wt2, row(bt2),
        row(bp), wp.reshape(hor, d + dc, E),
        row(b0), w0,
        ln_g.reshape(L, 1, H), ln_b.reshape(L, 1, H),
        wr, br.reshape(L, 1, H),
        row(lnf_g), row(lnf_b),
        wf, row(bf),
    ]

    def const_spec(a):
        return pl.BlockSpec(a.shape, lambda i: (0,) * a.ndim)

    weight_specs = [const_spec(a) for a in weight_inputs]
    any_spec = pl.BlockSpec(memory_space=pl.ANY)
    for k in (3, 6, 8, 11, 15):
        weight_specs[k] = any_spec

    in_specs = (
        [pl.BlockSpec((tb, 1), lambda i: (i, 0)),
         any_spec, any_spec]
        + weight_specs
    )

    body = functools.partial(_denoiser_body, num_layers=L, eps=1e-5,
                             tb=tb, d=d)
    y = pl.pallas_call(
        body,
        out_shape=jax.ShapeDtypeStruct((b_pad, dout), f32),
        grid=(b_pad // tb,),
        in_specs=in_specs,
        out_specs=pl.BlockSpec((tb, dout), lambda i: (i, 0)),
        scratch_shapes=[
            pltpu.VMEM((tb, hor * d), f32),
            pltpu.VMEM((tb, hor * dc), f32),
            pltpu.VMEM((E, E), f32),
            pltpu.VMEM((hor, d, E), f32),
            pltpu.VMEM((hor, dc, E), f32),
            pltpu.VMEM((E, H), f32),
            pltpu.VMEM((L, H, H), f32),
            pltpu.VMEM((H, dout), f32),
            pltpu.SemaphoreType.DMA((L + 7,)),
        ],
        compiler_params=pltpu.CompilerParams(
            dimension_semantics=("parallel",),
        ),
    )(t, trajf, actf, *weight_inputs)
    return y[:b].reshape(b, hor, d)


# speedup vs baseline: 1.0214x; 1.0214x over previous
import functools
import math

import jax
import jax.numpy as jnp
from jax.experimental import pallas as pl
from jax.experimental.pallas import tpu as pltpu


def _denoiser_body(
    t_ref, traj_hbm, act_hbm,
    fw_ref, wt1_ref, bt1_ref, wt2_hbm, bt2_ref,
    bp_ref, wp_hbm, b0_ref, w0_hbm,
    lng_ref, lnb_ref, wr_hbm, br_ref,
    lnfg_ref, lnfb_ref, wf_hbm, bfin_ref,
    out_ref,
    traj_scr, act_scr, wt2_scr, wpt_scr, wpa_scr, w0_scr,
    wr_scr, wf_scr, sems,
    *, num_layers: int, eps: float, tb: int, d: int,
):
    f32 = jnp.float32
    bf16 = jnp.bfloat16
    step = pl.program_id(0)
    rows = pl.ds(step * tb, tb)

    streams = [
        (wt2_hbm, wt2_scr),
        (traj_hbm.at[rows, :], traj_scr),
        (act_hbm.at[rows, :], act_scr),
        (wp_hbm.at[:, 0:d, :], wpt_scr),
        (wp_hbm.at[:, d:, :], wpa_scr),
        (w0_hbm, w0_scr),
    ] + [(wr_hbm.at[i], wr_scr.at[i]) for i in range(num_layers)] \
        + [(wf_hbm, wf_scr)]
    for k, (src, dst) in enumerate(streams):
        pltpu.make_async_copy(src, dst, sems.at[k]).start()

    def wait(k):
        dst = streams[k][1]
        pltpu.make_async_copy(dst, dst, sems.at[k]).wait()

    def mm(a, w_ref):
        w = w_ref[...]
        w2 = w.reshape(-1, w.shape[-1]) if w.ndim == 3 else w
        return jnp.dot(a.astype(bf16), w2.astype(bf16),
                       preferred_element_type=f32)

    def ln_relu(v, g, b):
        mu = jnp.mean(v, axis=-1, keepdims=True)
        m2 = jnp.mean(v * v, axis=-1, keepdims=True)
        var = jnp.maximum(m2 - mu * mu, 0.0)
        return jnp.maximum((v - mu) * jax.lax.rsqrt(var + eps) * g + b, 0.0)

    t = t_ref[...]
    phase = t * (fw_ref[0] * (2.0 * math.pi))
    sc = jnp.concatenate(
        [jnp.sin(phase), jnp.sin(phase + 0.5 * math.pi)], axis=1)
    h1 = (t * wt1_ref[0]
          + jnp.dot(sc, wt1_ref[1:, :], preferred_element_type=f32)
          + bt1_ref[...])
    h1 = h1 * (1.0 / (1.0 + jnp.exp(-h1)))
    wait(0)
    te = mm(h1, wt2_scr)

    wait(1), wait(2), wait(3), wait(4)
    zx = mm(traj_scr[...], wpt_scr) + mm(act_scr[...], wpa_scr)
    bpt = bp_ref[...] + bt2_ref[...]

    ns = 4 if tb % 4 == 0 else 1
    sb = tb // ns
    sl = [slice(s * sb, (s + 1) * sb) for s in range(ns)]
    wait(5)
    h = [None] * ns
    for s in range(ns):
        zs = zx[sl[s]] + bpt + te[sl[s]]
        h[s] = mm(zs, w0_scr) + b0_ref[...]
    a = [None] * ns
    for i in range(num_layers):
        wait(6 + i)
        a[0] = ln_relu(h[0], lng_ref[i], lnb_ref[i])
        for s in range(1, ns):
            a[s] = ln_relu(h[s], lng_ref[i], lnb_ref[i])
            h[s - 1] = h[s - 1] + mm(a[s - 1], wr_scr.at[i]) + br_ref[i]
        h[ns - 1] = h[ns - 1] + mm(a[ns - 1], wr_scr.at[i]) + br_ref[i]
    wait(6 + num_layers)
    a[0] = ln_relu(h[0], lnfg_ref[...], lnfb_ref[...])
    for s in range(1, ns):
        a[s] = ln_relu(h[s], lnfg_ref[...], lnfb_ref[...])
        out_ref[sl[s - 1], :] = mm(a[s - 1], wf_scr) + bfin_ref[...]
    out_ref[sl[ns - 1], :] = mm(a[ns - 1], wf_scr) + bfin_ref[...]


def kernel(traj, act, timesteps, fourier_w, wt1, bt1, wt2, bt2, wp, bp,
           w0, b0, ln_g, ln_b, wr, br, lnf_g, lnf_b, wf, bf):
    f32 = jnp.float32
    b, hor, d = traj.shape
    dc = act.shape[-1]
    trajf = traj.reshape(b, hor * d)
    actf = act.reshape(b, hor * dc)
    t = timesteps.reshape(b, 1)

    E = wt2.shape[0]
    H = w0.shape[1]
    L = wr.shape[0]
    dout = wf.shape[1]
    half = fourier_w.shape[0]

    tb = 1024 if b >= 1024 else max(8, ((b + 7) // 8) * 8)
    b_pad = ((b + tb - 1) // tb) * tb
    if b_pad != b:
        trajf = jnp.pad(trajf, ((0, b_pad - b), (0, 0)))
        actf = jnp.pad(actf, ((0, b_pad - b), (0, 0)))
        t = jnp.pad(t, ((0, b_pad - b), (0, 0)))

    def row(v):
        return v.reshape(1, -1)

    weight_inputs = [
        fourier_w.reshape(1, half),
        wt1, row(bt1),
        wt2, row(bt2),
        row(bp), wp.reshape(hor, d + dc, E),
        row(b0), w0,
        ln_g.reshape(L, 1, H), ln_b.reshape(L, 1, H),
        wr, br.reshape(L, 1, H),
        row(lnf_g), row(lnf_b),
        wf, row(bf),
    ]

    def const_spec(a):
        return pl.BlockSpec(a.shape, lambda i: (0,) * a.ndim)

    weight_specs = [const_spec(a) for a in weight_inputs]
    any_spec = pl.BlockSpec(memory_space=pl.ANY)
    for k in (3, 6, 8, 11, 15):
        weight_specs[k] = any_spec

    in_specs = (
        [pl.BlockSpec((tb, 1), lambda i: (i, 0)),
         any_spec, any_spec]
        + weight_specs
    )

    body = functools.partial(_denoiser_body, num_layers=L, eps=1e-5,
                             tb=tb, d=d)
    y = pl.pallas_call(
        body,
        out_shape=jax.ShapeDtypeStruct((b_pad, dout), f32),
        grid=(b_pad // tb,),
        in_specs=in_specs,
        out_specs=pl.BlockSpec((tb, dout), lambda i: (i, 0)),
        scratch_shapes=[
            pltpu.VMEM((tb, hor * d), f32),
            pltpu.VMEM((tb, hor * dc), f32),
            pltpu.VMEM((E, E), f32),
            pltpu.VMEM((hor, d, E), f32),
            pltpu.VMEM((hor, dc, E), f32),
            pltpu.VMEM((E, H), f32),
            pltpu.VMEM((L, H, H), f32),
            pltpu.VMEM((H, dout), f32),
            pltpu.SemaphoreType.DMA((L + 7,)),
        ],
        compiler_params=pltpu.CompilerParams(
            dimension_semantics=("parallel",),
        ),
    )(t, trajf, actf, *weight_inputs)
    return y[:b].reshape(b, hor, d)
